# Initial kernel scaffold; baseline (speedup 1.0000x reference)
import jax
import jax.numpy as jnp
from jax.experimental import pallas as pl


def kernel(x, seg, tok_embed, pos_embed, seg_embed, gamma, beta):
    raise NotImplementedError("write your pallas kernel here")


# speedup vs baseline: 6.0238x; 6.0238x over previous
import jax
import jax.numpy as jnp
from jax.experimental import pallas as pl
from jax.experimental.pallas import tpu as pltpu
from jax.experimental.pallas import tpu_sc as plsc

EPS = 1e-5


def _sc_gather(tok_embed, idx_flat, n, d):
    w = 128
    mesh = plsc.VectorSubcoreMesh(core_axis_name="c", subcore_axis_name="s")

    @pl.kernel(
        out_type=jax.ShapeDtypeStruct((n, d), tok_embed.dtype),
        mesh=mesh,
    )
    def gather_kernel(tok_hbm, i_hbm, o_hbm):
        def body(i_vmem, o_vmem):
            pltpu.sync_copy(tok_hbm.at[i_vmem.at[0]], o_vmem)

        pltpu.emit_pipeline(
            body,
            grid=(n // w,),
            in_specs=[pl.BlockSpec((1, w), lambda i: (0, i))],
            out_specs=[pl.BlockSpec((w, d), lambda i: (i, 0))],
            core_axis_name=("c", "s"),
            dimension_semantics=(pltpu.PARALLEL,),
        )(i_hbm, o_hbm)

    return gather_kernel(tok_embed, idx_flat)


def _tc_addln_body(g_ref, seg_ref, pos_ref, sege_ref, gamma_ref, beta_ref,
                   o_ref):
    g = g_ref[...]
    s = seg_ref[...]
    pos = pos_ref[...]
    e0 = sege_ref[0, :]
    e1 = sege_ref[1, :]
    segv = jnp.where(s[..., None] == 0, e0, e1)
    emb = g + pos[None, :, :] + segv
    mean = jnp.mean(emb, axis=-1, keepdims=True)
    cent = emb - mean
    var = jnp.mean(cent * cent, axis=-1, keepdims=True)
    normed = cent * jax.lax.rsqrt(var + EPS)
    o_ref[...] = normed * gamma_ref[0, :] + beta_ref[0, :]


def kernel(x, seg, tok_embed, pos_embed, seg_embed, gamma, beta):
    b, l = x.shape
    v, d = tok_embed.shape
    n = b * l

    idx_flat = x.reshape(1, n).astype(jnp.int32)
    gathered = _sc_gather(tok_embed, idx_flat, n, d).reshape(b, l, d)

    bb = 8
    out = pl.pallas_call(
        _tc_addln_body,
        grid=(b // bb,),
        in_specs=[
            pl.BlockSpec((bb, l, d), lambda i: (i, 0, 0)),
            pl.BlockSpec((bb, l), lambda i: (i, 0)),
            pl.BlockSpec((l, d), lambda i: (0, 0)),
            pl.BlockSpec((seg_embed.shape[0], d), lambda i: (0, 0)),
            pl.BlockSpec((1, d), lambda i: (0, 0)),
            pl.BlockSpec((1, d), lambda i: (0, 0)),
        ],
        out_specs=pl.BlockSpec((bb, l, d), lambda i: (i, 0, 0)),
        out_shape=jax.ShapeDtypeStruct((b, l, d), jnp.float32),
    )(gathered, seg, pos_embed, seg_embed, gamma.reshape(1, d),
      beta.reshape(1, d))
    return out


# speedup vs baseline: 6.7278x; 1.1169x over previous
import jax
import jax.numpy as jnp
from jax.experimental import pallas as pl
from jax.experimental.pallas import tpu as pltpu
from jax.experimental.pallas import tpu_sc as plsc

EPS = 1e-5


def _sc_gather(tok_embed, idx_flat, n, d):
    w = 128
    mesh = plsc.VectorSubcoreMesh(core_axis_name="c", subcore_axis_name="s")

    @pl.kernel(
        out_type=jax.ShapeDtypeStruct((n, d), tok_embed.dtype),
        mesh=mesh,
    )
    def gather_kernel(tok_hbm, i_hbm, o_hbm):
        def body(i_vmem, o_vmem):
            pltpu.sync_copy(tok_hbm.at[i_vmem.at[0]], o_vmem)

        pltpu.emit_pipeline(
            body,
            grid=(n // w,),
            in_specs=[pl.BlockSpec((1, w), lambda i: (0, i))],
            out_specs=[pl.BlockSpec((w, d), lambda i: (i, 0))],
            core_axis_name=("c", "s"),
            dimension_semantics=(pltpu.PARALLEL,),
        )(i_hbm, o_hbm)

    return gather_kernel(tok_embed, idx_flat)


def _tc_addln_body(g_ref, seg_ref, ps0_ref, gb_ref, o_ref):
    g = g_ref[...]
    sf = seg_ref[...].astype(jnp.float32)[..., None]
    ps0 = ps0_ref[...][None, :, :]
    dps = gb_ref[2, :]
    emb = g + ps0 + sf * dps
    d = emb.shape[-1]
    mean = jnp.sum(emb, axis=-1, keepdims=True) * (1.0 / d)
    sq = jnp.sum(emb * emb, axis=-1, keepdims=True) * (1.0 / d)
    var = sq - mean * mean
    k = jax.lax.rsqrt(var + EPS)
    gamma = gb_ref[0, :]
    beta = gb_ref[1, :]
    o_ref[...] = (emb - mean) * (k * gamma) + beta


def kernel(x, seg, tok_embed, pos_embed, seg_embed, gamma, beta):
    b, l = x.shape
    v, d = tok_embed.shape
    n = b * l

    idx_flat = x.reshape(1, n).astype(jnp.int32)
    gathered = _sc_gather(tok_embed, idx_flat, n, d).reshape(b, l, d)

    ps0 = pos_embed + seg_embed[0]
    dps = seg_embed[1] - seg_embed[0]
    gb = jnp.stack([gamma, beta, dps])

    bb = 16
    out = pl.pallas_call(
        _tc_addln_body,
        grid=(b // bb,),
        in_specs=[
            pl.BlockSpec((bb, l, d), lambda i: (i, 0, 0)),
            pl.BlockSpec((bb, l), lambda i: (i, 0)),
            pl.BlockSpec((l, d), lambda i: (0, 0)),
            pl.BlockSpec((3, d), lambda i: (0, 0)),
        ],
        out_specs=pl.BlockSpec((bb, l, d), lambda i: (i, 0, 0)),
        out_shape=jax.ShapeDtypeStruct((b, l, d), jnp.float32),
    )(gathered, seg, ps0, gb)
    return out


# speedup vs baseline: 7.2504x; 1.0777x over previous
import jax
import jax.numpy as jnp
from jax.experimental import pallas as pl
from jax.experimental.pallas import tpu as pltpu
from jax.experimental.pallas import tpu_sc as plsc

EPS = 1e-5


def _sc_gather(tok_embed, idx_flat, n, d):
    w = 128
    mesh = plsc.VectorSubcoreMesh(core_axis_name="c", subcore_axis_name="s")

    @pl.kernel(
        out_type=jax.ShapeDtypeStruct((n, d), tok_embed.dtype),
        mesh=mesh,
    )
    def gather_kernel(tok_hbm, i_hbm, o_hbm):
        def body(i_vmem, o_vmem):
            pltpu.sync_copy(tok_hbm.at[i_vmem.at[0]], o_vmem)

        pltpu.emit_pipeline(
            body,
            grid=(n // w,),
            in_specs=[pl.BlockSpec((1, w), lambda i: (0, i))],
            out_specs=[pl.BlockSpec((w, d), lambda i: (i, 0))],
            core_axis_name=("c", "s"),
            dimension_semantics=(pltpu.PARALLEL,),
        )(i_hbm, o_hbm)

    return gather_kernel(tok_embed, idx_flat)


def _tc_addln_body(g_ref, seg_ref, ps0_ref, gb_ref, ones_ref, o_ref):
    bb, l, d = g_ref.shape
    g = g_ref[...]
    sf = seg_ref[...].astype(jnp.float32)[..., None]
    ps0 = ps0_ref[...][None, :, :]
    dps = gb_ref[2, :]
    emb = (g + ps0 + sf * dps).reshape(bb * l, d)
    ones = ones_ref[...]
    sums = jax.lax.dot_general(
        emb.astype(jnp.bfloat16), ones, (((1,), (0,)), ((), ())),
        preferred_element_type=jnp.float32)
    sqsums = jax.lax.dot_general(
        (emb * emb).astype(jnp.bfloat16), ones, (((1,), (0,)), ((), ())),
        preferred_element_type=jnp.float32)
    mean = sums * (1.0 / d)
    var = sqsums * (1.0 / d) - mean * mean
    k = jax.lax.rsqrt(var + EPS)
    gamma = gb_ref[0, :]
    beta = gb_ref[1, :]
    out = (emb - mean) * (k * gamma) + beta
    o_ref[...] = out.reshape(bb, l, d)


def kernel(x, seg, tok_embed, pos_embed, seg_embed, gamma, beta):
    b, l = x.shape
    v, d = tok_embed.shape
    n = b * l

    idx_flat = x.reshape(1, n).astype(jnp.int32)
    gathered = _sc_gather(tok_embed, idx_flat, n, d).reshape(b, l, d)

    ps0 = pos_embed + seg_embed[0]
    dps = seg_embed[1] - seg_embed[0]
    gb = jnp.stack([gamma, beta, dps])

    bb = 16
    out = pl.pallas_call(
        _tc_addln_body,
        grid=(b // bb,),
        in_specs=[
            pl.BlockSpec((bb, l, d), lambda i: (i, 0, 0)),
            pl.BlockSpec((bb, l), lambda i: (i, 0)),
            pl.BlockSpec((l, d), lambda i: (0, 0)),
            pl.BlockSpec((3, d), lambda i: (0, 0)),
            pl.BlockSpec((d, d), lambda i: (0, 0)),
        ],
        out_specs=pl.BlockSpec((bb, l, d), lambda i: (i, 0, 0)),
        out_shape=jax.ShapeDtypeStruct((b, l, d), jnp.float32),
    )(gathered, seg, ps0, gb, jnp.ones((d, d), jnp.bfloat16))
    return out


# speedup vs baseline: 8.5169x; 1.1747x over previous
import jax
import jax.numpy as jnp
from jax.experimental import pallas as pl
from jax.experimental.pallas import tpu as pltpu
from jax.experimental.pallas import tpu_sc as plsc

EPS = 1e-5


def _sc_gather(table, idx_flat, n):
    w = 128
    d = table.shape[1]
    mesh = plsc.VectorSubcoreMesh(core_axis_name="c", subcore_axis_name="s")

    @pl.kernel(
        out_type=jax.ShapeDtypeStruct((n, d), table.dtype),
        mesh=mesh,
    )
    def gather_kernel(tok_hbm, i_hbm, o_hbm):
        def body(i_vmem, o_vmem):
            pltpu.sync_copy(tok_hbm.at[i_vmem.at[0]], o_vmem)

        pltpu.emit_pipeline(
            body,
            grid=(n // w,),
            in_specs=[pl.BlockSpec((1, w), lambda i: (0, i))],
            out_specs=[pl.BlockSpec((w, d), lambda i: (i, 0))],
            core_axis_name=("c", "s"),
            dimension_semantics=(pltpu.PARALLEL,),
        )(i_hbm, o_hbm)

    return gather_kernel(table, idx_flat)


def _tc_addln_body(g_ref, seg_ref, ps0_ref, gb_ref, ones_ref, *rest):
    o_ref = rest[-1]
    bb, l, d = g_ref.shape
    g = g_ref[...]
    sf = seg_ref[...].astype(jnp.float32)[..., None]
    ps0 = ps0_ref[...][None, :, :]
    dps = gb_ref[2, :]
    emb = (g + ps0 + sf * dps).reshape(bb * l, d)
    ones = ones_ref[...]
    sums = jax.lax.dot_general(
        emb.astype(jnp.bfloat16), ones, (((1,), (0,)), ((), ())),
        preferred_element_type=jnp.float32)
    sqsums = jax.lax.dot_general(
        (emb * emb).astype(jnp.bfloat16), ones, (((1,), (0,)), ((), ())),
        preferred_element_type=jnp.float32)
    mean = sums * (1.0 / d)
    var = sqsums * (1.0 / d) - mean * mean
    k = jax.lax.rsqrt(var + EPS)
    gamma = gb_ref[0, :]
    beta = gb_ref[1, :]
    out = (emb - mean) * (k * gamma) + beta
    o_ref[...] = out.reshape(bb, l, d)


def kernel(x, seg, tok_embed, pos_embed, seg_embed, gamma, beta):
    b, l = x.shape
    v, d = tok_embed.shape

    ps0 = pos_embed + seg_embed[0]
    dps = seg_embed[1] - seg_embed[0]
    gb = jnp.stack([gamma, beta, dps])
    ones = jnp.ones((d, d), jnp.bfloat16)

    nchunks = 4
    bc = b // nchunks
    bb = 16
    spc = bc // bb
    idx = x.reshape(nchunks, 1, bc * l).astype(jnp.int32)

    out = None
    for c in range(nchunks):
        g_c = _sc_gather(tok_embed, idx[c], bc * l).reshape(bc, l, d)
        common = dict(
            grid=(spc,),
            out_shape=jax.ShapeDtypeStruct((b, l, d), jnp.float32),
            out_specs=pl.BlockSpec(
                (bb, l, d), lambda i, c=c: (i + c * spc, 0, 0)),
        )
        in_specs = [
            pl.BlockSpec((bb, l, d), lambda i: (i, 0, 0)),
            pl.BlockSpec((bb, l), lambda i, c=c: (i + c * spc, 0)),
            pl.BlockSpec((l, d), lambda i: (0, 0)),
            pl.BlockSpec((3, d), lambda i: (0, 0)),
            pl.BlockSpec((d, d), lambda i: (0, 0)),
        ]
        if out is None:
            out = pl.pallas_call(
                _tc_addln_body, in_specs=in_specs, **common,
            )(g_c, seg, ps0, gb, ones)
        else:
            out = pl.pallas_call(
                _tc_addln_body,
                in_specs=in_specs + [
                    pl.BlockSpec(memory_space=pl.MemorySpace.ANY)],
                input_output_aliases={5: 0},
                **common,
            )(g_c, seg, ps0, gb, ones, out)
    return out
